# Initial kernel scaffold; baseline (speedup 1.0000x reference)
import functools

import jax
import jax.numpy as jnp
from jax import lax
from jax.experimental import pallas as pl
from jax.experimental.pallas import tpu as pltpu
from jax.experimental.pallas import tpu_sc as plsc

NC = 2
NS = 16
TILES = NC * NS


def _sc_partials(x, srcr, dstr, *, n, d, c_chunks, k):
    zr = 125
    stripe = n // NS

    @functools.partial(
        pl.kernel,
        out_type=jax.ShapeDtypeStruct((NC, n, d), jnp.float32),
        mesh=plsc.VectorSubcoreMesh(core_axis_name="c", subcore_axis_name="s"),
        scratch_types=[
            pltpu.VMEM((c_chunks, k), jnp.int32),
            pltpu.VMEM((c_chunks, k), jnp.int32),
            pltpu.VMEM((k, d), jnp.float32),
            pltpu.VMEM((k, d), jnp.float32),
            pltpu.VMEM((zr, d), jnp.float32),
            pltpu.VMEM_SHARED((n, d), jnp.float32),
            pltpu.SemaphoreType.DMA,
            pltpu.SemaphoreType.DMA,
        ],
    )
    def sc_kernel(x_hbm, srcr_hbm, dstr_hbm, out_hbm,
                  sidx, didx, rows0, rows1, zbuf, acc, sem0, sem1):
        c = lax.axis_index("c")
        s = lax.axis_index("s")
        t = c * NS + s

        pltpu.sync_copy(srcr_hbm.at[t], sidx)
        pltpu.sync_copy(dstr_hbm.at[t], didx)

        z16 = jnp.zeros((16,), jnp.float32)

        @pl.loop(0, zr)
        def _(r):
            @pl.loop(0, d // 16)
            def _(q):
                zbuf[r, pl.ds(q * 16, 16)] = z16

        @pl.loop(0, stripe // zr)
        def _(i):
            pltpu.sync_copy(zbuf, acc.at[pl.ds(s * stripe + i * zr, zr)])

        plsc.subcore_barrier()

        def start_gather(j, rows, sem):
            pltpu.async_copy(x_hbm.at[sidx.at[j]], rows, sem)

        def wait_gather(j, rows, sem):
            pltpu.make_async_copy(x_hbm.at[sidx.at[j]], rows, sem).wait()

        start_gather(0, rows0, sem0)
        start_gather(1, rows1, sem1)

        @pl.loop(0, c_chunks, step=2)
        def _(j):
            wait_gather(j, rows0, sem0)
            pltpu.sync_copy(rows0, acc.at[didx.at[j]], add=True)

            @pl.when(j + 2 < c_chunks)
            def _():
                start_gather(j + 2, rows0, sem0)

            @pl.when(j + 1 < c_chunks)
            def _():
                wait_gather(j + 1, rows1, sem1)
                pltpu.sync_copy(rows1, acc.at[didx.at[j + 1]], add=True)

                @pl.when(j + 3 < c_chunks)
                def _():
                    start_gather(j + 3, rows1, sem1)

        plsc.subcore_barrier()

        pltpu.sync_copy(acc.at[pl.ds(s * stripe, stripe)],
                        out_hbm.at[c].at[pl.ds(s * stripe, stripe)])

    return sc_kernel(x, srcr, dstr)


def _tc_add_body(a_ref, b_ref, o_ref):
    o_ref[...] = a_ref[...] + b_ref[...]


def kernel(x, edge_index):
    n, d = x.shape
    e = edge_index.shape[1]
    k = 80
    c_chunks = e // (TILES * k)

    src = edge_index[0].reshape(TILES, c_chunks, k)
    dst = edge_index[1].reshape(TILES, c_chunks, k)

    partial = _sc_partials(x, src, dst, n=n, d=d, c_chunks=c_chunks, k=k)

    return pl.pallas_call(
        _tc_add_body,
        out_shape=jax.ShapeDtypeStruct((n, d), jnp.float32),
    )(partial[0], partial[1])


# speedup vs baseline: 3.4849x; 3.4849x over previous
import functools

import jax
import jax.numpy as jnp
from jax import lax
from jax.experimental import pallas as pl
from jax.experimental.pallas import tpu as pltpu
from jax.experimental.pallas import tpu_sc as plsc

NC = 2
NS = 16
TILES = NC * NS
K = 128
BPT = 80
IB = 16
NB = BPT // IB


def _sc_partials(x, srcp, dstp, zrows, *, n_pad, d):
    stripe = n_pad // NS

    @functools.partial(
        pl.kernel,
        out_type=jax.ShapeDtypeStruct((NC, n_pad, d), jnp.float32),
        mesh=plsc.VectorSubcoreMesh(core_axis_name="c", subcore_axis_name="s"),
        scratch_types=[
            pltpu.VMEM((IB, K), jnp.int32),
            pltpu.VMEM((IB, K), jnp.int32),
            pltpu.VMEM((K, d), jnp.float32),
            pltpu.VMEM((K, d), jnp.float32),
            pltpu.VMEM_SHARED((n_pad, d), jnp.float32),
            pltpu.SemaphoreType.DMA,
            pltpu.SemaphoreType.DMA,
        ],
    )
    def sc_kernel(x_hbm, srcp_hbm, dstp_hbm, zrows_hbm, out_hbm,
                  sidx, didx, rows0, rows1, acc, sem0, sem1):
        c = lax.axis_index("c")
        s = lax.axis_index("s")
        t = c * NS + s

        pltpu.sync_copy(zrows_hbm, acc.at[pl.ds(s * stripe, stripe)])
        plsc.subcore_barrier()

        def start_gather(j, rows, sem):
            pltpu.async_copy(x_hbm.at[sidx.at[j]], rows, sem)

        def wait_gather(j, rows, sem):
            pltpu.make_async_copy(x_hbm.at[sidx.at[j]], rows, sem).wait()

        @pl.loop(0, NB)
        def _(b):
            base = t * BPT + b * IB
            pltpu.sync_copy(srcp_hbm.at[pl.ds(base, IB)], sidx)
            pltpu.sync_copy(dstp_hbm.at[pl.ds(base, IB)], didx)

            start_gather(0, rows0, sem0)
            start_gather(1, rows1, sem1)

            @pl.loop(0, IB, step=2)
            def _(j):
                wait_gather(j, rows0, sem0)
                pltpu.sync_copy(rows0, acc.at[didx.at[j]], add=True)

                @pl.when(j + 2 < IB)
                def _():
                    start_gather(j + 2, rows0, sem0)

                wait_gather(j + 1, rows1, sem1)
                pltpu.sync_copy(rows1, acc.at[didx.at[j + 1]], add=True)

                @pl.when(j + 3 < IB)
                def _():
                    start_gather(j + 3, rows1, sem1)

        plsc.subcore_barrier()

        pltpu.sync_copy(acc.at[pl.ds(s * stripe, stripe)],
                        out_hbm.at[c].at[pl.ds(s * stripe, stripe)])

    return sc_kernel(x, srcp, dstp, zrows)


def _tc_add_body(a_ref, b_ref, o_ref):
    o_ref[...] = a_ref[...] + b_ref[...]


def kernel(x, edge_index):
    n, d = x.shape
    e = edge_index.shape[1]
    n_pad = ((n + NS * 8 - 1) // (NS * 8)) * (NS * 8)
    e_pad = TILES * BPT * K
    pad = e_pad - e

    src = jnp.concatenate(
        [edge_index[0], jnp.zeros((pad,), jnp.int32)]).reshape(-1, K)
    dst = jnp.concatenate(
        [edge_index[1], jnp.full((pad,), n, jnp.int32)]).reshape(-1, K)
    zrows = jnp.zeros((n_pad // NS, d), jnp.float32)

    partial = _sc_partials(x, src, dst, zrows, n_pad=n_pad, d=d)

    padded = pl.pallas_call(
        _tc_add_body,
        out_shape=jax.ShapeDtypeStruct((n_pad, d), jnp.float32),
    )(partial[0], partial[1])
    return padded[:n]


# speedup vs baseline: 3.4851x; 1.0000x over previous
import functools

import jax
import jax.numpy as jnp
from jax import lax
from jax.experimental import pallas as pl
from jax.experimental.pallas import tpu as pltpu
from jax.experimental.pallas import tpu_sc as plsc

NC = 2
NS = 16
TILES = NC * NS
K = 128
BPT = 80
IB = 16
NB = BPT // IB


def _sc_partials(x, srcp, dstp, zrows, *, n_pad, d):
    stripe = n_pad // NS

    @functools.partial(
        pl.kernel,
        out_type=jax.ShapeDtypeStruct((NC, n_pad, d), jnp.float32),
        mesh=plsc.VectorSubcoreMesh(core_axis_name="c", subcore_axis_name="s"),
        scratch_types=[
            pltpu.VMEM((IB, K), jnp.int32),
            pltpu.VMEM((IB, K), jnp.int32),
            pltpu.VMEM((K, d), jnp.float32),
            pltpu.VMEM((K, d), jnp.float32),
            pltpu.VMEM_SHARED((n_pad, d), jnp.float32),
            pltpu.SemaphoreType.DMA,
            pltpu.SemaphoreType.DMA,
        ],
    )
    def sc_kernel(x_hbm, srcp_hbm, dstp_hbm, zrows_hbm, out_hbm,
                  sidx, didx, rows0, rows1, acc, sem0, sem1):
        c = lax.axis_index("c")
        s = lax.axis_index("s")
        t = c * NS + s

        pltpu.sync_copy(zrows_hbm, acc.at[pl.ds(s * stripe, stripe)])
        plsc.subcore_barrier()

        def start_gather(j, rows, sem):
            pltpu.async_copy(x_hbm.at[sidx.at[j]], rows, sem)

        def wait_gather(j, rows, sem):
            pltpu.make_async_copy(x_hbm.at[sidx.at[j]], rows, sem).wait()

        @pl.loop(0, NB)
        def _(b):
            base = t * BPT + b * IB
            pltpu.sync_copy(srcp_hbm.at[pl.ds(base, IB)], sidx)
            pltpu.sync_copy(dstp_hbm.at[pl.ds(base, IB)], didx)

            start_gather(0, rows0, sem0)
            start_gather(1, rows1, sem1)

            @pl.loop(0, IB, step=2)
            def _(j):
                wait_gather(j, rows0, sem0)
                pltpu.sync_copy(rows0, acc.at[didx.at[j]], add=True)

                @pl.when(j + 2 < IB)
                def _():
                    start_gather(j + 2, rows0, sem0)

                wait_gather(j + 1, rows1, sem1)
                pltpu.sync_copy(rows1, acc.at[didx.at[j + 1]], add=True)

                @pl.when(j + 3 < IB)
                def _():
                    start_gather(j + 3, rows1, sem1)

        plsc.subcore_barrier()

        pltpu.sync_copy(acc.at[pl.ds(s * stripe, stripe)],
                        out_hbm.at[c].at[pl.ds(s * stripe, stripe)])

    return sc_kernel(x, srcp, dstp, zrows)


def _tc_add_body(a_ref, b_ref, o_ref):
    o_ref[...] = a_ref[...] + b_ref[...]


def kernel(x, edge_index):
    n, d = x.shape
    e = edge_index.shape[1]
    n_pad = ((n + NS * 8 - 1) // (NS * 8)) * (NS * 8)
    e_pad = TILES * BPT * K
    pad = e_pad - e

    dummy_dst = n + jnp.arange(pad, dtype=jnp.int32) % (n_pad - n)
    src = jnp.concatenate(
        [edge_index[0], jnp.zeros((pad,), jnp.int32)]).reshape(-1, K)
    dst = jnp.concatenate(
        [edge_index[1], dummy_dst]).reshape(-1, K)
    zrows = jnp.zeros((n_pad // NS, d), jnp.float32)

    partial = _sc_partials(x, src, dst, zrows, n_pad=n_pad, d=d)

    padded = pl.pallas_call(
        _tc_add_body,
        out_shape=jax.ShapeDtypeStruct((n_pad, d), jnp.float32),
    )(partial[0], partial[1])
    return padded[:n]


# speedup vs baseline: 11.4318x; 3.2802x over previous
import functools

import jax
import jax.numpy as jnp
from jax import lax
from jax.experimental import pallas as pl
from jax.experimental.pallas import tpu as pltpu
from jax.experimental.pallas import tpu_sc as plsc

NC = 2
NS = 16
TILES = NC * NS
K = 128
BPT = 80
IB = 16
NB = BPT // IB


def _sc_partials(x, srcp, dstp, zrows, *, n_pad, d):
    stripe = n_pad // NS

    @functools.partial(
        pl.kernel,
        out_type=jax.ShapeDtypeStruct((NC, n_pad, d), jnp.float32),
        mesh=plsc.VectorSubcoreMesh(core_axis_name="c", subcore_axis_name="s"),
        scratch_types=[
            pltpu.VMEM((IB, K), jnp.int32),
            pltpu.VMEM((IB, K), jnp.int32),
            pltpu.VMEM((K, d), jnp.float32),
            pltpu.VMEM((K, d), jnp.float32),
            pltpu.VMEM_SHARED((n_pad, d), jnp.float32),
            pltpu.SemaphoreType.DMA,
            pltpu.SemaphoreType.DMA,
        ],
    )
    def sc_kernel(x_hbm, srcp_hbm, dstp_hbm, zrows_hbm, out_hbm,
                  sidx, didx, rows0, rows1, acc, sem0, sem1):
        c = lax.axis_index("c")
        s = lax.axis_index("s")
        t = c * NS + s

        pltpu.sync_copy(zrows_hbm, acc.at[pl.ds(s * stripe, stripe)])
        plsc.subcore_barrier()

        def start_gather(j, rows, sem):
            pltpu.async_copy(x_hbm.at[sidx.at[j]], rows, sem)

        def wait_gather(j, rows, sem):
            pltpu.make_async_copy(x_hbm.at[sidx.at[j]], rows, sem).wait()

        @pl.loop(0, NB)
        def _(b):
            base = t * BPT + b * IB
            pltpu.sync_copy(srcp_hbm.at[pl.ds(base, IB)], sidx)
            pltpu.sync_copy(dstp_hbm.at[pl.ds(base, IB)], didx)

            start_gather(0, rows0, sem0)
            start_gather(1, rows1, sem1)

            @pl.loop(0, IB, step=2)
            def _(j):
                wait_gather(j, rows0, sem0)
                pltpu.sync_copy(rows0, acc.at[didx.at[j]], add=True)

                @pl.when(j + 2 < IB)
                def _():
                    start_gather(j + 2, rows0, sem0)

                wait_gather(j + 1, rows1, sem1)
                pltpu.sync_copy(rows1, acc.at[didx.at[j + 1]], add=True)

                @pl.when(j + 3 < IB)
                def _():
                    start_gather(j + 3, rows1, sem1)

        plsc.subcore_barrier()

        pltpu.sync_copy(acc.at[pl.ds(s * stripe, stripe)],
                        out_hbm.at[c].at[pl.ds(s * stripe, stripe)])

    return sc_kernel(x, srcp, dstp, zrows)


def _tc_add_body(a_ref, b_ref, o_ref):
    o_ref[...] = a_ref[...] + b_ref[...]


def kernel(x, edge_index):
    n, d = x.shape
    e = edge_index.shape[1]
    n_pad = ((n + NS * 8 - 1) // (NS * 8)) * (NS * 8)
    e_pad = TILES * BPT * K
    pad = e_pad - e

    dummy_dst = n + jnp.arange(pad, dtype=jnp.int32) % (n_pad - n)
    dummy_src = jnp.arange(pad, dtype=jnp.int32) % n
    src = jnp.concatenate(
        [edge_index[0], dummy_src]).reshape(-1, K)
    dst = jnp.concatenate(
        [edge_index[1], dummy_dst]).reshape(-1, K)
    zrows = jnp.zeros((n_pad // NS, d), jnp.float32)

    partial = _sc_partials(x, src, dst, zrows, n_pad=n_pad, d=d)

    padded = pl.pallas_call(
        _tc_add_body,
        out_shape=jax.ShapeDtypeStruct((n_pad, d), jnp.float32),
    )(partial[0], partial[1])
    return padded[:n]


# speedup vs baseline: 12.2155x; 1.0686x over previous
import functools

import jax
import jax.numpy as jnp
from jax import lax
from jax.experimental import pallas as pl
from jax.experimental.pallas import tpu as pltpu
from jax.experimental.pallas import tpu_sc as plsc

NC = 2
NS = 16
TILES = NC * NS
K = 128
BPT = 80
IB = 16
NB = BPT // IB


def _sc_partials(x, srcp, dstp, zrows, *, n_pad, d):
    stripe = n_pad // NS

    @functools.partial(
        pl.kernel,
        out_type=jax.ShapeDtypeStruct((NC, n_pad, d), jnp.float32),
        mesh=plsc.VectorSubcoreMesh(core_axis_name="c", subcore_axis_name="s"),
        scratch_types=[
            pltpu.VMEM((IB, K), jnp.int32),
            pltpu.VMEM((IB, K), jnp.int32),
            pltpu.VMEM((K, d), jnp.float32),
            pltpu.VMEM((K, d), jnp.float32),
            pltpu.VMEM_SHARED((n_pad, d), jnp.float32),
            pltpu.SemaphoreType.DMA,
            pltpu.SemaphoreType.DMA,
        ],
    )
    def sc_kernel(x_hbm, srcp_hbm, dstp_hbm, zrows_hbm, out_hbm,
                  sidx, didx, rows0, rows1, acc, sem0, sem1):
        c = lax.axis_index("c")
        s = lax.axis_index("s")
        t = c * NS + s

        pltpu.sync_copy(zrows_hbm, acc.at[pl.ds(s * stripe, stripe)])
        plsc.subcore_barrier()

        def start_gather(j, rows, sem):
            pltpu.async_copy(x_hbm.at[sidx.at[j]], rows, sem)

        def wait_gather(j, rows, sem):
            pltpu.make_async_copy(x_hbm.at[sidx.at[j]], rows, sem).wait()

        @pl.loop(0, NB)
        def _(b):
            base = t * BPT + b * IB
            pltpu.sync_copy(srcp_hbm.at[pl.ds(base, IB)], sidx)
            pltpu.sync_copy(dstp_hbm.at[pl.ds(base, IB)], didx)

            start_gather(0, rows0, sem0)
            start_gather(1, rows1, sem1)

            @pl.loop(0, IB, step=2)
            def _(j):
                wait_gather(j, rows0, sem0)
                pltpu.sync_copy(rows0, acc.at[didx.at[j]], add=True)

                @pl.when(j + 2 < IB)
                def _():
                    start_gather(j + 2, rows0, sem0)

                wait_gather(j + 1, rows1, sem1)
                pltpu.sync_copy(rows1, acc.at[didx.at[j + 1]], add=True)

                @pl.when(j + 3 < IB)
                def _():
                    start_gather(j + 3, rows1, sem1)

        plsc.subcore_barrier()

        pltpu.sync_copy(acc.at[pl.ds(s * stripe, stripe)],
                        out_hbm.at[c].at[pl.ds(s * stripe, stripe)])

    return sc_kernel(x, srcp, dstp, zrows)


def _tc_add_body(p_ref, o_ref):
    n = o_ref.shape[0]
    o_ref[...] = p_ref[0, :n, :] + p_ref[1, :n, :]


def kernel(x, edge_index):
    n, d = x.shape
    e = edge_index.shape[1]
    n_pad = ((n + NS * 8 - 1) // (NS * 8)) * (NS * 8)
    e_pad = TILES * BPT * K
    pad = e_pad - e

    dummy_dst = n + jnp.arange(pad, dtype=jnp.int32) % (n_pad - n)
    dummy_src = jnp.arange(pad, dtype=jnp.int32) % n
    src = jnp.concatenate(
        [edge_index[0], dummy_src]).reshape(-1, K)
    dst = jnp.concatenate(
        [edge_index[1], dummy_dst]).reshape(-1, K)
    zrows = jnp.zeros((n_pad // NS, d), jnp.float32)

    partial = _sc_partials(x, src, dst, zrows, n_pad=n_pad, d=d)

    return pl.pallas_call(
        _tc_add_body,
        out_shape=jax.ShapeDtypeStruct((n, d), jnp.float32),
    )(partial)
